# baseline (speedup 1.0000x reference)
import functools

import jax
import jax.numpy as jnp
from jax import lax
from jax.experimental import pallas as pl
from jax.experimental.pallas import tpu as pltpu
from jax.experimental.pallas import tpu_sc as plsc

N = 10000
E = 320000
D = 128
CA = 144
NC = 2
NS = 16
NW = NC * NS
EW = E // NW
CH = 80
NITER = EW // CH
NP_ = 10240
RPT = NP_ // NS
ZR = 128
LANES = 16

_mesh = lambda: plsc.VectorSubcoreMesh(core_axis_name="c", subcore_axis_name="s")


@functools.partial(
    pl.kernel,
    out_type=jax.ShapeDtypeStruct((NC, NP_, CA), jnp.float32),
    mesh=_mesh(),
    scratch_types=[
        pltpu.VMEM((2, 2, CH), jnp.int32),
        pltpu.VMEM((2, CH, CA), jnp.float32),
        pltpu.VMEM_SHARED((NP_, CA), jnp.float32),
        pltpu.SemaphoreType.DMA,
        pltpu.SemaphoreType.DMA,
        pltpu.SemaphoreType.DMA,
        pltpu.SemaphoreType.DMA,
    ],
    compiler_params=pltpu.CompilerParams(use_tc_tiling_on_sc=False,
                                         needs_layout_passes=False),
)
def _aggregate(xa_hbm, ei_hbm, out_hbm, eidx, rows, table,
               gsem0, gsem1, ssem0, ssem1):
    c = lax.axis_index("c")
    s = lax.axis_index("s")
    wid = c * NS + s
    gsem = (gsem0, gsem1)
    ssem = (ssem0, ssem1)

    def zb(i, carry):
        for j in range(CA // LANES):
            rows[0, i, pl.ds(j * LANES, LANES)] = jnp.zeros((LANES,),
                                                            jnp.float32)
        return carry

    lax.fori_loop(0, CH, zb, 0)
    for t in range(RPT // CH):
        pltpu.sync_copy(rows.at[0], table.at[pl.ds(s * RPT + t * CH, CH)])
    plsc.subcore_barrier()

    def idxcopy(cc, b):
        pltpu.sync_copy(ei_hbm.at[:, pl.ds(wid * EW + cc * CH, CH)],
                        eidx.at[b])

    def gather(cc, b):
        return pltpu.make_async_copy(xa_hbm.at[eidx.at[b, 0]], rows.at[b],
                                     gsem[b])

    def scatter(b):
        return pltpu.make_async_copy(rows.at[b], table.at[eidx.at[b, 1]],
                                     ssem[b])

    idxcopy(0, 0)
    gather(0, 0).start()

    @pl.loop(0, NITER, step=2)
    def _visits(j):
        for b in range(2):
            cc = j + b
            nb = 1 - b

            @pl.when(cc < NITER)
            def _():
                @pl.when(cc + 1 < NITER)
                def _():
                    @pl.when(cc >= 1)
                    def _():
                        scatter(nb).wait()

                    idxcopy(cc + 1, nb)
                    gather(cc + 1, nb).start()

                gather(cc, b).wait()
                pltpu.async_copy(rows.at[b], table.at[eidx.at[b, 1]],
                                 ssem[b], add=True)

    scatter((NITER - 2) % 2).wait()
    scatter((NITER - 1) % 2).wait()
    plsc.subcore_barrier()

    for t in range(RPT // ZR):
        r0 = s * RPT + t * ZR
        pltpu.sync_copy(table.at[pl.ds(r0, ZR)], out_hbm.at[c, pl.ds(r0, ZR)])


_BLK = 1000


def _dense_body(p0, p1, wgnn, bgnn, w1a, w1b, b1, a_out, b_out):
    agg = p0[:, :D] + p1[:, :D]
    deg = p0[:, D:D + 1] + p1[:, D:D + 1]
    h = agg / jnp.maximum(deg, 1.0)
    h = jnp.maximum(
        jnp.dot(h, wgnn[...], preferred_element_type=jnp.float32) + bgnn[...],
        0.0)
    a_out[...] = jnp.dot(h, w1a[...], preferred_element_type=jnp.float32) + b1[...]
    b_out[...] = jnp.dot(h, w1b[...], preferred_element_type=jnp.float32)


def _dense(p0, p1, wgnn, bgnn, w1a, w1b, b1):
    full = lambda i: (0, 0)
    blk = lambda i: (i, 0)
    return pl.pallas_call(
        _dense_body,
        grid=(N // _BLK,),
        in_specs=[
            pl.BlockSpec((_BLK, CA), blk),
            pl.BlockSpec((_BLK, CA), blk),
            pl.BlockSpec((D, D), full),
            pl.BlockSpec((1, D), full),
            pl.BlockSpec((D, D), full),
            pl.BlockSpec((D, D), full),
            pl.BlockSpec((1, D), full),
        ],
        out_specs=[
            pl.BlockSpec((_BLK, D), blk),
            pl.BlockSpec((_BLK, D), blk),
        ],
        out_shape=[
            jax.ShapeDtypeStruct((N, D), jnp.float32),
            jax.ShapeDtypeStruct((N, D), jnp.float32),
        ],
    )(p0, p1, wgnn, bgnn, w1a, w1b, b1)


@functools.partial(
    pl.kernel,
    out_type=jax.ShapeDtypeStruct((E,), jnp.float32),
    mesh=_mesh(),
    scratch_types=[
        pltpu.VMEM((3, CH), jnp.int32),
        pltpu.VMEM((3, CH), jnp.int32),
        pltpu.VMEM((3, CH, D), jnp.float32),
        pltpu.VMEM((3, CH, D), jnp.float32),
        pltpu.VMEM((D,), jnp.float32),
        pltpu.VMEM((LANES,), jnp.float32),
        pltpu.VMEM((3, CH), jnp.float32),
        pltpu.SemaphoreType.DMA,
        pltpu.SemaphoreType.DMA,
        pltpu.SemaphoreType.DMA,
        pltpu.SemaphoreType.DMA,
        pltpu.SemaphoreType.DMA,
        pltpu.SemaphoreType.DMA,
        pltpu.SemaphoreType.DMA,
        pltpu.SemaphoreType.DMA,
        pltpu.SemaphoreType.DMA,
    ],
    compiler_params=pltpu.CompilerParams(needs_layout_passes=False),
)
def _edge_mlp(a_hbm, b_hbm, src_hbm, dst_hbm, w2_hbm, b2v_hbm, out_hbm,
              sidx, didx, arows, brows, w2v, b2v, outbuf,
              gsa0, gsa1, gsa2, gsb0, gsb1, gsb2, osem0, osem1, osem2):
    c = lax.axis_index("c")
    s = lax.axis_index("s")
    wid = c * NS + s
    gsa = (gsa0, gsa1, gsa2)
    gsb = (gsb0, gsb1, gsb2)
    osem = (osem0, osem1, osem2)
    pltpu.sync_copy(w2_hbm, w2v)
    pltpu.sync_copy(b2v_hbm, b2v)

    def idxcopy(cc, b):
        base = wid * EW + cc * CH
        pltpu.sync_copy(src_hbm.at[pl.ds(base, CH)], sidx.at[b])
        pltpu.sync_copy(dst_hbm.at[pl.ds(base, CH)], didx.at[b])

    def gathers(b):
        return (pltpu.make_async_copy(a_hbm.at[sidx.at[b]], arows.at[b],
                                      gsa[b]),
                pltpu.make_async_copy(b_hbm.at[didx.at[b]], brows.at[b],
                                      gsb[b]))

    def outcopy(cc, b):
        return pltpu.make_async_copy(
            outbuf.at[b], out_hbm.at[pl.ds(wid * EW + cc * CH, CH)], osem[b])

    for p in range(2):
        idxcopy(p, p)
        ga, gb = gathers(p)
        ga.start()
        gb.start()
    lane = lax.iota(jnp.int32, LANES)
    w2regs = [w2v[pl.ds(u * LANES, LANES)] for u in range(D // LANES)]
    b2reg = b2v[...]

    @pl.loop(0, NITER, step=3)
    def _visits(j):
        for b in range(3):
            cc = j + b
            nb = (b + 2) % 3

            @pl.when(cc < NITER)
            def _():
                @pl.when(cc + 2 < NITER)
                def _():
                    @pl.when(cc >= 1)
                    def _():
                        outcopy(cc - 1, nb).wait()

                    idxcopy(cc + 2, nb)
                    ga, gb = gathers(nb)
                    ga.start()
                    gb.start()

                ga, gb = gathers(b)
                ga.wait()
                gb.wait()

                def group(g, carry2):
                    def edge16(k, vec):
                        e = g * LANES + k
                        acc = b2reg
                        for u in range(D // LANES):
                            av = arows[b, e, pl.ds(u * LANES, LANES)]
                            bv = brows[b, e, pl.ds(u * LANES, LANES)]
                            acc = acc + jnp.maximum(av + bv, 0.0) * w2regs[u]
                        return jnp.where(lane == k, jnp.sum(acc), vec)

                    vec = lax.fori_loop(0, LANES, edge16,
                                        jnp.zeros((LANES,), jnp.float32))
                    outbuf[b, pl.ds(g * LANES, LANES)] = vec
                    return carry2

                lax.fori_loop(0, CH // LANES, group, 0)
                outcopy(cc, b).start()

    outcopy(NITER - 3, (NITER - 3) % 3).wait()
    outcopy(NITER - 2, (NITER - 2) % 3).wait()
    outcopy(NITER - 1, (NITER - 1) % 3).wait()


def kernel(x, edge_index, W_gnn, b_gnn, W1, b1, W2, b2):
    ei = edge_index.astype(jnp.int32)
    xa = jnp.concatenate(
        [x, jnp.ones((N, 1), x.dtype), jnp.zeros((N, CA - D - 1), x.dtype)],
        axis=1)
    parts = _aggregate(xa, ei)
    a_tab, b_tab = _dense(parts[0], parts[1], W_gnn, b_gnn.reshape(1, D),
                          W1[:D], W1[D:], b1.reshape(1, D))
    b2v = jnp.broadcast_to(b2 / LANES, (LANES,)).astype(jnp.float32)
    out = _edge_mlp(a_tab, b_tab, ei[0], ei[1], W2.reshape(D), b2v)
    return out.reshape(E, 1)


# speedup vs baseline: 1.2689x; 1.2689x over previous
import functools

import jax
import jax.numpy as jnp
from jax import lax
from jax.experimental import pallas as pl
from jax.experimental.pallas import tpu as pltpu
from jax.experimental.pallas import tpu_sc as plsc

N = 10000
E = 320000
D = 128
CA = 144
NC = 2
NS = 16
NW = NC * NS
EW = E // NW
CH = 80
NITER = EW // CH
NP_ = 10240
RPT = NP_ // NS
ZR = 128
LANES = 16

_mesh = lambda: plsc.VectorSubcoreMesh(core_axis_name="c", subcore_axis_name="s")


@functools.partial(
    pl.kernel,
    out_type=jax.ShapeDtypeStruct((NC, NP_, CA), jnp.float32),
    mesh=_mesh(),
    scratch_types=[
        pltpu.VMEM((4, 2, CH), jnp.int32),
        pltpu.VMEM((2, CH, CA), jnp.float32),
        pltpu.VMEM_SHARED((NP_, CA), jnp.float32),
        pltpu.SemaphoreType.DMA,
        pltpu.SemaphoreType.DMA,
        pltpu.SemaphoreType.DMA,
        pltpu.SemaphoreType.DMA,
        pltpu.SemaphoreType.DMA,
        pltpu.SemaphoreType.DMA,
        pltpu.SemaphoreType.DMA,
        pltpu.SemaphoreType.DMA,
    ],
    compiler_params=pltpu.CompilerParams(use_tc_tiling_on_sc=False,
                                         needs_layout_passes=False),
)
def _aggregate(xa_hbm, ei_hbm, out_hbm, eidx, rows, table,
               gsem0, gsem1, ssem0, ssem1, isem0, isem1, isem2, isem3):
    c = lax.axis_index("c")
    s = lax.axis_index("s")
    wid = c * NS + s
    gsem = (gsem0, gsem1)
    ssem = (ssem0, ssem1)
    isem = (isem0, isem1, isem2, isem3)

    def zb(i, carry):
        for j in range(CA // LANES):
            rows[0, i, pl.ds(j * LANES, LANES)] = jnp.zeros((LANES,),
                                                            jnp.float32)
        return carry

    lax.fori_loop(0, CH, zb, 0)
    for t in range(RPT // CH):
        pltpu.sync_copy(rows.at[0], table.at[pl.ds(s * RPT + t * CH, CH)])
    plsc.subcore_barrier()

    def idxcopy(cc, ib):
        return pltpu.make_async_copy(
            ei_hbm.at[:, pl.ds(wid * EW + cc * CH, CH)], eidx.at[ib],
            isem[ib])

    def gather(db, ib):
        return pltpu.make_async_copy(xa_hbm.at[eidx.at[ib, 0]], rows.at[db],
                                     gsem[db])

    def scatter(db, ib):
        return pltpu.make_async_copy(rows.at[db], table.at[eidx.at[ib, 1]],
                                     ssem[db])

    idxcopy(0, 0).start()
    idxcopy(1, 1).start()
    idxcopy(0, 0).wait()
    gather(0, 0).start()

    @pl.loop(0, NITER, step=4)
    def _visits(j):
        for b in range(4):
            cc = j + b
            db = b % 2
            nd = 1 - db
            ib = b
            ni = (b + 1) % 4
            pf = (b + 2) % 4

            @pl.when(cc < NITER)
            def _():
                @pl.when(cc + 2 < NITER)
                def _():
                    idxcopy(cc + 2, pf).start()

                @pl.when(cc + 1 < NITER)
                def _():
                    @pl.when(cc >= 1)
                    def _():
                        scatter(nd, ni).wait()

                    idxcopy(cc + 1, ni).wait()
                    gather(nd, ni).start()

                gather(db, ib).wait()
                pltpu.async_copy(rows.at[db], table.at[eidx.at[ib, 1]],
                                 ssem[db], add=True)

    scatter((NITER - 2) % 2, (NITER - 2) % 4).wait()
    scatter((NITER - 1) % 2, (NITER - 1) % 4).wait()
    plsc.subcore_barrier()

    for t in range(RPT // ZR):
        r0 = s * RPT + t * ZR
        pltpu.sync_copy(table.at[pl.ds(r0, ZR)], out_hbm.at[c, pl.ds(r0, ZR)])


_BLK = 1000


def _dense_body(p0, p1, wgnn, bgnn, w1a, w1b, b1, a_out, b_out):
    agg = p0[:, :D] + p1[:, :D]
    deg = p0[:, D:D + 1] + p1[:, D:D + 1]
    h = agg / jnp.maximum(deg, 1.0)
    h = jnp.maximum(
        jnp.dot(h, wgnn[...], preferred_element_type=jnp.float32) + bgnn[...],
        0.0)
    a_out[...] = jnp.dot(h, w1a[...], preferred_element_type=jnp.float32) + b1[...]
    b_out[...] = jnp.dot(h, w1b[...], preferred_element_type=jnp.float32)


def _dense(p0, p1, wgnn, bgnn, w1a, w1b, b1):
    full = lambda i: (0, 0)
    blk = lambda i: (i, 0)
    return pl.pallas_call(
        _dense_body,
        grid=(N // _BLK,),
        in_specs=[
            pl.BlockSpec((_BLK, CA), blk),
            pl.BlockSpec((_BLK, CA), blk),
            pl.BlockSpec((D, D), full),
            pl.BlockSpec((1, D), full),
            pl.BlockSpec((D, D), full),
            pl.BlockSpec((D, D), full),
            pl.BlockSpec((1, D), full),
        ],
        out_specs=[
            pl.BlockSpec((_BLK, D), blk),
            pl.BlockSpec((_BLK, D), blk),
        ],
        out_shape=[
            jax.ShapeDtypeStruct((N, D), jnp.float32),
            jax.ShapeDtypeStruct((N, D), jnp.float32),
        ],
    )(p0, p1, wgnn, bgnn, w1a, w1b, b1)


@functools.partial(
    pl.kernel,
    out_type=jax.ShapeDtypeStruct((E,), jnp.float32),
    mesh=_mesh(),
    scratch_types=[
        pltpu.VMEM((3, 2, CH), jnp.int32),
        pltpu.VMEM((3, CH, D), jnp.float32),
        pltpu.VMEM((3, CH, D), jnp.float32),
        pltpu.VMEM((D,), jnp.float32),
        pltpu.VMEM((LANES,), jnp.float32),
        pltpu.VMEM((3, CH), jnp.float32),
        pltpu.SemaphoreType.DMA,
        pltpu.SemaphoreType.DMA,
        pltpu.SemaphoreType.DMA,
        pltpu.SemaphoreType.DMA,
        pltpu.SemaphoreType.DMA,
        pltpu.SemaphoreType.DMA,
        pltpu.SemaphoreType.DMA,
        pltpu.SemaphoreType.DMA,
        pltpu.SemaphoreType.DMA,
        pltpu.SemaphoreType.DMA,
        pltpu.SemaphoreType.DMA,
        pltpu.SemaphoreType.DMA,
    ],
    compiler_params=pltpu.CompilerParams(use_tc_tiling_on_sc=False,
                                         needs_layout_passes=False),
)
def _edge_mlp(a_hbm, b_hbm, ei_hbm, w2_hbm, b2v_hbm, out_hbm,
              eidx, arows, brows, w2v, b2v, outbuf,
              gsa0, gsa1, gsa2, gsb0, gsb1, gsb2, osem0, osem1, osem2,
              isem0, isem1, isem2):
    c = lax.axis_index("c")
    s = lax.axis_index("s")
    wid = c * NS + s
    gsa = (gsa0, gsa1, gsa2)
    gsb = (gsb0, gsb1, gsb2)
    osem = (osem0, osem1, osem2)
    isem = (isem0, isem1, isem2)
    pltpu.sync_copy(w2_hbm, w2v)
    pltpu.sync_copy(b2v_hbm, b2v)

    def idxcopy(cc, b):
        return pltpu.make_async_copy(
            ei_hbm.at[:, pl.ds(wid * EW + cc * CH, CH)], eidx.at[b],
            isem[b])

    def gathers(b):
        return (pltpu.make_async_copy(a_hbm.at[eidx.at[b, 0]], arows.at[b],
                                      gsa[b]),
                pltpu.make_async_copy(b_hbm.at[eidx.at[b, 1]], brows.at[b],
                                      gsb[b]))

    def outcopy(cc, b):
        return pltpu.make_async_copy(
            outbuf.at[b], out_hbm.at[pl.ds(wid * EW + cc * CH, CH)], osem[b])

    idxcopy(0, 0).start()
    idxcopy(1, 1).start()
    idxcopy(0, 0).wait()
    ga, gb = gathers(0)
    ga.start()
    gb.start()
    lane = lax.iota(jnp.int32, LANES)
    w2regs = [w2v[pl.ds(u * LANES, LANES)] for u in range(D // LANES)]
    b2reg = b2v[...]

    @pl.loop(0, NITER, step=3)
    def _visits(j):
        for b in range(3):
            cc = j + b
            nb = (b + 2) % 3
            ni = (b + 1) % 3

            @pl.when(cc < NITER)
            def _():
                @pl.when(cc + 2 < NITER)
                def _():
                    @pl.when(cc >= 1)
                    def _():
                        outcopy(cc - 1, nb).wait()

                    idxcopy(cc + 2, nb).start()

                @pl.when(cc + 1 < NITER)
                def _():
                    idxcopy(cc + 1, ni).wait()
                    ga, gb = gathers(ni)
                    ga.start()
                    gb.start()

                ga, gb = gathers(b)
                ga.wait()
                gb.wait()

                def group(g, carry2):
                    def edge16(k, vec):
                        e = g * LANES + k
                        acc = b2reg
                        for u in range(D // LANES):
                            av = arows[b, e, pl.ds(u * LANES, LANES)]
                            bv = brows[b, e, pl.ds(u * LANES, LANES)]
                            acc = acc + jnp.maximum(av + bv, 0.0) * w2regs[u]
                        return jnp.where(lane == k, jnp.sum(acc), vec)

                    vec = lax.fori_loop(0, LANES, edge16,
                                        jnp.zeros((LANES,), jnp.float32))
                    outbuf[b, pl.ds(g * LANES, LANES)] = vec
                    return carry2

                lax.fori_loop(0, CH // LANES, group, 0)
                outcopy(cc, b).start()

    outcopy(NITER - 3, (NITER - 3) % 3).wait()
    outcopy(NITER - 2, (NITER - 2) % 3).wait()
    outcopy(NITER - 1, (NITER - 1) % 3).wait()


def kernel(x, edge_index, W_gnn, b_gnn, W1, b1, W2, b2):
    ei = edge_index.astype(jnp.int32)
    xa = jnp.concatenate(
        [x, jnp.ones((N, 1), x.dtype), jnp.zeros((N, CA - D - 1), x.dtype)],
        axis=1)
    parts = _aggregate(xa, ei)
    a_tab, b_tab = _dense(parts[0], parts[1], W_gnn, b_gnn.reshape(1, D),
                          W1[:D], W1[D:], b1.reshape(1, D))
    b2v = jnp.broadcast_to(b2 / LANES, (LANES,)).astype(jnp.float32)
    out = _edge_mlp(a_tab, b_tab, ei, W2.reshape(D), b2v)
    return out.reshape(E, 1)
